# baseline (speedup 1.0000x reference)
import functools
import math

import jax
import jax.numpy as jnp
from jax import lax
from jax.experimental import pallas as pl
from jax.experimental.pallas import tpu as pltpu
from jax.experimental.pallas import tpu_sc as plsc

D_MODEL = 64
SCALE = math.sqrt(D_MODEL)
CHUNK = 128


@functools.lru_cache(maxsize=None)
def _make_sc_lookup(V: int, N: int, D: int):
    info = plsc.get_sparse_core_info()
    NC, NS, L = info.num_cores, info.num_subcores, info.num_lanes
    NW = NC * NS
    assert N % (NW * CHUNK) == 0
    per_w = N // NW
    n_chunks = per_w // CHUNK
    assert D % L == 0

    mesh = plsc.VectorSubcoreMesh(core_axis_name="c", subcore_axis_name="s")

    @functools.partial(
        pl.kernel,
        mesh=mesh,
        out_type=jax.ShapeDtypeStruct((N, D), jnp.float32),
        scratch_types=[
            pltpu.VMEM((n_chunks, CHUNK), jnp.int32),
            pltpu.VMEM((CHUNK, D), jnp.float32),
            pltpu.SemaphoreType.DMA,
        ],
        compiler_params=pltpu.CompilerParams(use_tc_tiling_on_sc=False),
    )
    def lookup(x_hbm, table_hbm, out_hbm, idx_v, rows_v, sem):
        wid = lax.axis_index("s") * NC + lax.axis_index("c")
        row_base = wid * per_w
        chunk_base = wid * n_chunks
        pltpu.sync_copy(x_hbm.at[pl.ds(chunk_base, n_chunks)], idx_v)

        def body(j, carry):
            pltpu.async_copy(table_hbm.at[idx_v.at[j]], rows_v, sem).wait()

            def scale_row(r, c2):
                for k in range(D // L):
                    sl = pl.ds(k * L, L)
                    rows_v[r, sl] = rows_v[r, sl] * SCALE
                return c2

            lax.fori_loop(0, CHUNK, scale_row, 0)
            pltpu.sync_copy(
                rows_v, out_hbm.at[pl.ds(row_base + j * CHUNK, CHUNK)]
            )
            return carry

        lax.fori_loop(0, n_chunks, body, 0)

    return lookup


def kernel(x, table):
    B, L = x.shape
    V, D = table.shape
    N = B * L
    xf = x.reshape(N // CHUNK, CHUNK).astype(jnp.int32)
    out = _make_sc_lookup(V, N, D)(xf, table)
    return out.reshape(B, L, D)


# speedup vs baseline: 1.0988x; 1.0988x over previous
import functools
import math

import jax
import jax.numpy as jnp
from jax import lax
from jax.experimental import pallas as pl
from jax.experimental.pallas import tpu as pltpu
from jax.experimental.pallas import tpu_sc as plsc

D_MODEL = 64
SCALE = math.sqrt(D_MODEL)
CHUNK = 128
NBUF = 4


@functools.lru_cache(maxsize=None)
def _make_sc_lookup(V: int, N: int, D: int):
    info = plsc.get_sparse_core_info()
    NC, NS, L = info.num_cores, info.num_subcores, info.num_lanes
    NW = NC * NS
    assert N % (NW * CHUNK) == 0
    per_w = N // NW
    n_chunks = per_w // CHUNK
    assert n_chunks % NBUF == 0
    assert D % L == 0

    mesh = plsc.VectorSubcoreMesh(core_axis_name="c", subcore_axis_name="s")

    @functools.partial(
        pl.kernel,
        mesh=mesh,
        out_type=jax.ShapeDtypeStruct((N, D), jnp.float32),
        scratch_types=[
            pltpu.VMEM((n_chunks, CHUNK), jnp.int32),
            pltpu.VMEM((NBUF, CHUNK, D), jnp.float32),
            pltpu.VMEM((NBUF, CHUNK, D), jnp.float32),
        ]
        + [pltpu.SemaphoreType.DMA] * (2 * NBUF),
        compiler_params=pltpu.CompilerParams(use_tc_tiling_on_sc=False),
    )
    def lookup(x_hbm, table_hbm, out_hbm, idx_v, gbuf, sbuf, *sems):
        gsems = sems[:NBUF]
        ssems = sems[NBUF:]
        wid = lax.axis_index("s") * NC + lax.axis_index("c")
        row_base = wid * per_w
        chunk_base = wid * n_chunks
        pltpu.sync_copy(x_hbm.at[pl.ds(chunk_base, n_chunks)], idx_v)

        for b in range(NBUF):
            pltpu.async_copy(table_hbm.at[idx_v.at[b]], gbuf.at[b], gsems[b])

        def outer(t, carry):
            g0 = t * NBUF
            for b in range(NBUF):
                j = g0 + b
                pltpu.make_async_copy(
                    table_hbm.at[idx_v.at[j]], gbuf.at[b], gsems[b]
                ).wait()

                @pl.when(t > 0)
                def _drain():
                    pltpu.make_async_copy(
                        sbuf.at[b],
                        out_hbm.at[pl.ds(row_base, CHUNK)],
                        ssems[b],
                    ).wait()

                def scale_row(r, c):
                    for k in range(D // L):
                        sl = pl.ds(k * L, L)
                        sbuf[b, r, sl] = gbuf[b, r, sl] * SCALE
                    return c

                lax.fori_loop(0, CHUNK, scale_row, 0, unroll=4)

                @pl.when(j + NBUF < n_chunks)
                def _refill():
                    pltpu.async_copy(
                        table_hbm.at[idx_v.at[j + NBUF]], gbuf.at[b], gsems[b]
                    )

                pltpu.async_copy(
                    sbuf.at[b],
                    out_hbm.at[pl.ds(row_base + j * CHUNK, CHUNK)],
                    ssems[b],
                )
            return carry

        lax.fori_loop(0, n_chunks // NBUF, outer, 0)

        for b in range(NBUF):
            pltpu.make_async_copy(
                sbuf.at[b], out_hbm.at[pl.ds(row_base, CHUNK)], ssems[b]
            ).wait()

    return lookup


def kernel(x, table):
    B, L = x.shape
    V, D = table.shape
    N = B * L
    xf = x.reshape(N // CHUNK, CHUNK).astype(jnp.int32)
    out = _make_sc_lookup(V, N, D)(xf, table)
    return out.reshape(B, L, D)
